# Initial kernel scaffold; baseline (speedup 1.0000x reference)
import jax
import jax.numpy as jnp
from jax.experimental import pallas as pl


def kernel(features, edge_index, W1, b1, W2, b2, W_out, b_out):
    raise NotImplementedError("write your pallas kernel here")


# speedup vs baseline: 3.0083x; 3.0083x over previous
import dataclasses
import functools

import jax
import jax.numpy as jnp
from jax import lax
from jax.experimental import pallas as pl
from jax.experimental.pallas import tpu as pltpu
from jax.experimental.pallas import tpu_sc as plsc

_N = 10000
_E = 160000
_D = 256
_H = 128
_NPAD = 10240
_EPAD = 163840
_NC = 2
_NS = 16
_NW = _NC * _NS
_CHUNK = 128
_VEC = 16
_BLK = 1024


def _mesh():
    return plsc.VectorSubcoreMesh(core_axis_name="c", subcore_axis_name="s")


def _sc_params():
    cp = pltpu.CompilerParams()
    if "needs_layout_passes" in pltpu.CompilerParams.__dataclass_fields__:
        cp = dataclasses.replace(cp, needs_layout_passes=False)
    return cp


@functools.cache
def _deg_kernel():
    @functools.partial(
        pl.kernel,
        out_type=(jax.ShapeDtypeStruct((_NW, _NPAD), jnp.float32),
                  jax.ShapeDtypeStruct((_NW, _NPAD), jnp.float32)),
        mesh=_mesh(),
        compiler_params=_sc_params(),
        scratch_types=[
            pltpu.VMEM((_NPAD,), jnp.float32),
            pltpu.VMEM((_NPAD,), jnp.float32),
            pltpu.VMEM((_CHUNK,), jnp.int32),
            pltpu.VMEM((_CHUNK,), jnp.int32),
        ],
    )
    def deg(src_hbm, dst_hbm, osrc_hbm, odst_hbm, hs, hd, bs, bd):
        wid = lax.axis_index("s") * _NC + lax.axis_index("c")
        zeros = jnp.zeros((_VEC,), jnp.float32)
        ones = jnp.ones((_VEC,), jnp.float32)

        @pl.loop(0, _NPAD, step=_VEC)
        def _(i):
            hs[pl.ds(i, _VEC)] = zeros
            hd[pl.ds(i, _VEC)] = zeros

        base = wid * (_EPAD // _NW)

        @pl.loop(0, _EPAD // _NW, step=_CHUNK)
        def _(j):
            pltpu.sync_copy(src_hbm.at[pl.ds(base + j, _CHUNK)], bs)
            pltpu.sync_copy(dst_hbm.at[pl.ds(base + j, _CHUNK)], bd)

            @pl.loop(0, _CHUNK, step=_VEC)
            def _(k):
                plsc.addupdate_scatter(hs, [bs[pl.ds(k, _VEC)]], ones)
                plsc.addupdate_scatter(hd, [bd[pl.ds(k, _VEC)]], ones)

        pltpu.sync_copy(hs, osrc_hbm.at[wid])
        pltpu.sync_copy(hd, odst_hbm.at[wid])

    return deg


@functools.cache
def _conv_kernel():
    rows_per_sub = _NPAD // _NS

    @functools.partial(
        pl.kernel,
        out_type=(jax.ShapeDtypeStruct((_NPAD, _H), jnp.float32),
                  jax.ShapeDtypeStruct((_NPAD, _H), jnp.float32)),
        mesh=_mesh(),
        compiler_params=_sc_params(),
        scratch_types=[
            pltpu.VMEM_SHARED((_NPAD, _H), jnp.float32),
            pltpu.VMEM((_CHUNK,), jnp.int32),
            pltpu.VMEM((_CHUNK,), jnp.int32),
            pltpu.VMEM((_CHUNK, _H), jnp.float32),
            pltpu.SemaphoreType.DMA,
        ],
    )
    def conv(tlo_hbm, thi_hbm, src_hbm, dst_hbm, olo_hbm, ohi_hbm,
             acc, sidx, didx, rows, sem):
        c = lax.axis_index("c")
        s = lax.axis_index("s")
        zeros = jnp.zeros((_VEC,), jnp.float32)

        @pl.loop(0, _CHUNK)
        def _(i):
            @pl.loop(0, _H, step=_VEC)
            def _(k):
                rows[i, pl.ds(k, _VEC)] = zeros

        @pl.loop(0, rows_per_sub, step=_CHUNK)
        def _(r):
            pltpu.sync_copy(rows, acc.at[pl.ds(s * rows_per_sub + r, _CHUNK)])

        plsc.subcore_barrier()

        ebase = s * (_EPAD // _NS)

        @pl.loop(0, _EPAD // _NS, step=_CHUNK)
        def _(j):
            pltpu.sync_copy(src_hbm.at[pl.ds(ebase + j, _CHUNK)], sidx)
            pltpu.sync_copy(dst_hbm.at[pl.ds(ebase + j, _CHUNK)], didx)

            @pl.when(c == 0)
            def _():
                pltpu.async_copy(tlo_hbm.at[sidx], rows, sem).wait()

            @pl.when(c == 1)
            def _():
                pltpu.async_copy(thi_hbm.at[sidx], rows, sem).wait()

            pltpu.sync_copy(rows, acc.at[didx], add=True)

        plsc.subcore_barrier()

        out_slice = pl.ds(s * rows_per_sub, rows_per_sub)

        @pl.when(c == 0)
        def _():
            pltpu.sync_copy(acc.at[out_slice], olo_hbm.at[out_slice])

        @pl.when(c == 1)
        def _():
            pltpu.sync_copy(acc.at[out_slice], ohi_hbm.at[out_slice])

    return conv


@functools.cache
def _edge_kernel():
    @functools.partial(
        pl.kernel,
        out_type=jax.ShapeDtypeStruct((_EPAD,), jnp.float32),
        mesh=_mesh(),
        compiler_params=_sc_params(),
        scratch_types=[
            pltpu.VMEM((_NPAD,), jnp.float32),
            pltpu.VMEM((_NPAD,), jnp.float32),
            pltpu.VMEM((_CHUNK,), jnp.int32),
            pltpu.VMEM((_CHUNK,), jnp.int32),
            pltpu.VMEM((_CHUNK,), jnp.float32),
        ],
    )
    def edge(p_hbm, q_hbm, src_hbm, dst_hbm, o_hbm, pv, qv, bs, bd, ov):
        wid = lax.axis_index("s") * _NC + lax.axis_index("c")
        pltpu.sync_copy(p_hbm, pv)
        pltpu.sync_copy(q_hbm, qv)
        base = wid * (_EPAD // _NW)

        @pl.loop(0, _EPAD // _NW, step=_CHUNK)
        def _(j):
            pltpu.sync_copy(src_hbm.at[pl.ds(base + j, _CHUNK)], bs)
            pltpu.sync_copy(dst_hbm.at[pl.ds(base + j, _CHUNK)], bd)

            @pl.loop(0, _CHUNK, step=_VEC)
            def _(k):
                z = (plsc.load_gather(pv, [bs[pl.ds(k, _VEC)]])
                     + plsc.load_gather(qv, [bd[pl.ds(k, _VEC)]]))
                ov[pl.ds(k, _VEC)] = 20.0 / (1.0 + jnp.exp(-z)) - 1.0

            pltpu.sync_copy(ov, o_hbm.at[pl.ds(base + j, _CHUNK)])

    return edge


def _stage_norms(fpad, hsrc, hdst):
    def body(f_ref, hs_ref, hd_ref, tlo_ref, thi_ref, ns_ref, nd_ref):
        od = jnp.sum(hs_ref[...], axis=0)
        idg = jnp.sum(hd_ref[...], axis=0)
        ns = jnp.where(od > 0, od, 1.0) ** -0.5
        nd = jnp.where(idg > 0, idg, 1.0) ** -0.5
        t = f_ref[...] * ns[:, None]
        tlo_ref[...] = t[:, :_H]
        thi_ref[...] = t[:, _H:]
        ns_ref[...] = ns[:, None]
        nd_ref[...] = nd[:, None]

    grid = _NPAD // _BLK
    return pl.pallas_call(
        body,
        grid=(grid,),
        in_specs=[
            pl.BlockSpec((_BLK, _D), lambda i: (i, 0)),
            pl.BlockSpec((_NW, _BLK), lambda i: (0, i)),
            pl.BlockSpec((_NW, _BLK), lambda i: (0, i)),
        ],
        out_specs=[
            pl.BlockSpec((_BLK, _H), lambda i: (i, 0)),
            pl.BlockSpec((_BLK, _H), lambda i: (i, 0)),
            pl.BlockSpec((_BLK, 1), lambda i: (i, 0)),
            pl.BlockSpec((_BLK, 1), lambda i: (i, 0)),
        ],
        out_shape=[
            jax.ShapeDtypeStruct((_NPAD, _H), jnp.float32),
            jax.ShapeDtypeStruct((_NPAD, _H), jnp.float32),
            jax.ShapeDtypeStruct((_NPAD, 1), jnp.float32),
            jax.ShapeDtypeStruct((_NPAD, 1), jnp.float32),
        ],
    )(fpad, hsrc, hdst)


def _stage_mid(alo, ahi, ns, nd, W, b8):
    def body(alo_ref, ahi_ref, ns_ref, nd_ref, w_ref, b_ref, olo_ref, ohi_ref):
        ndv = nd_ref[...]
        xlo = alo_ref[...] * ndv
        xhi = ahi_ref[...] * ndv
        y = (jnp.dot(xlo, w_ref[:_H, :], preferred_element_type=jnp.float32,
                     precision=lax.Precision.HIGHEST)
             + jnp.dot(xhi, w_ref[_H:, :], preferred_element_type=jnp.float32,
                       precision=lax.Precision.HIGHEST))
        h = jnp.maximum(y + b_ref[0:1, :], 0.0)
        t = h * ns_ref[...]
        olo_ref[...] = t[:, :_H]
        ohi_ref[...] = t[:, _H:]

    grid = _NPAD // _BLK
    return pl.pallas_call(
        body,
        grid=(grid,),
        in_specs=[
            pl.BlockSpec((_BLK, _H), lambda i: (i, 0)),
            pl.BlockSpec((_BLK, _H), lambda i: (i, 0)),
            pl.BlockSpec((_BLK, 1), lambda i: (i, 0)),
            pl.BlockSpec((_BLK, 1), lambda i: (i, 0)),
            pl.BlockSpec((_D, _D), lambda i: (0, 0)),
            pl.BlockSpec((8, _D), lambda i: (0, 0)),
        ],
        out_specs=[
            pl.BlockSpec((_BLK, _H), lambda i: (i, 0)),
            pl.BlockSpec((_BLK, _H), lambda i: (i, 0)),
        ],
        out_shape=[
            jax.ShapeDtypeStruct((_NPAD, _H), jnp.float32),
            jax.ShapeDtypeStruct((_NPAD, _H), jnp.float32),
        ],
    )(alo, ahi, ns, nd, W, b8)


def _stage_out(alo, ahi, nd, W, b8, wpq, bout8):
    def body(alo_ref, ahi_ref, nd_ref, w_ref, b_ref, wpq_ref, bo_ref,
             op_ref, oq_ref):
        ndv = nd_ref[...]
        xlo = alo_ref[...] * ndv
        xhi = ahi_ref[...] * ndv
        y = (jnp.dot(xlo, w_ref[:_H, :], preferred_element_type=jnp.float32,
                     precision=lax.Precision.HIGHEST)
             + jnp.dot(xhi, w_ref[_H:, :], preferred_element_type=jnp.float32,
                       precision=lax.Precision.HIGHEST))
        h = jnp.maximum(y + b_ref[0:1, :], 0.0)
        pq = jnp.dot(h, wpq_ref[...], preferred_element_type=jnp.float32,
                     precision=lax.Precision.HIGHEST)
        op_ref[...] = pq[:, 0:1] + bo_ref[0:1, 0:1]
        oq_ref[...] = pq[:, 1:2]

    grid = _NPAD // _BLK
    return pl.pallas_call(
        body,
        grid=(grid,),
        in_specs=[
            pl.BlockSpec((_BLK, _H), lambda i: (i, 0)),
            pl.BlockSpec((_BLK, _H), lambda i: (i, 0)),
            pl.BlockSpec((_BLK, 1), lambda i: (i, 0)),
            pl.BlockSpec((_D, _D), lambda i: (0, 0)),
            pl.BlockSpec((8, _D), lambda i: (0, 0)),
            pl.BlockSpec((_D, 2), lambda i: (0, 0)),
            pl.BlockSpec((8, 128), lambda i: (0, 0)),
        ],
        out_specs=[
            pl.BlockSpec((_BLK, 1), lambda i: (i, 0)),
            pl.BlockSpec((_BLK, 1), lambda i: (i, 0)),
        ],
        out_shape=[
            jax.ShapeDtypeStruct((_NPAD, 1), jnp.float32),
            jax.ShapeDtypeStruct((_NPAD, 1), jnp.float32),
        ],
    )(alo, ahi, nd, W, b8, wpq, bout8)


def kernel(features, edge_index, W1, b1, W2, b2, W_out, b_out):
    src = edge_index[0]
    dst = edge_index[1]
    pad = jnp.full((_EPAD - _E,), _N, dtype=jnp.int32)
    srcp = jnp.concatenate([src, pad])
    dstp = jnp.concatenate([dst, pad])
    fpad = jnp.pad(features, ((0, _NPAD - _N), (0, 0)))

    b1_8 = jnp.broadcast_to(b1[None, :], (8, _D))
    b2_8 = jnp.broadcast_to(b2[None, :], (8, _D))
    wpq = jnp.concatenate([W_out[:_D], W_out[_D:]], axis=1)
    bout8 = jnp.broadcast_to(b_out.reshape(1, 1), (8, 128))

    hsrc, hdst = _deg_kernel()(srcp, dstp)
    t1lo, t1hi, ns, nd = _stage_norms(fpad, hsrc, hdst)
    a1lo, a1hi = _conv_kernel()(t1lo, t1hi, srcp, dstp)
    t2lo, t2hi = _stage_mid(a1lo, a1hi, ns, nd, W1, b1_8)
    a2lo, a2hi = _conv_kernel()(t2lo, t2hi, srcp, dstp)
    p, q = _stage_out(a2lo, a2hi, nd, W2, b2_8, wpq, bout8)

    out = _edge_kernel()(p.reshape(_NPAD), q.reshape(_NPAD), srcp, dstp)
    return out[:_E, None]


# speedup vs baseline: 5.1466x; 1.7108x over previous
import dataclasses
import functools

import jax
import jax.numpy as jnp
from jax import lax
from jax.experimental import pallas as pl
from jax.experimental.pallas import tpu as pltpu
from jax.experimental.pallas import tpu_sc as plsc

_N = 10000
_E = 160000
_D = 256
_H = 128
_NPAD = 10240
_EPAD = 163840
_NC = 2
_NS = 16
_NW = _NC * _NS
_CHUNK = 128
_VEC = 16
_BLK = 1024
_EROWS = _EPAD // _CHUNK
_CPS = _EPAD // _NS // _CHUNK
_CPW = _EPAD // _NW // _CHUNK


def _mesh():
    return plsc.VectorSubcoreMesh(core_axis_name="c", subcore_axis_name="s")


def _sc_params():
    cp = pltpu.CompilerParams()
    if "needs_layout_passes" in pltpu.CompilerParams.__dataclass_fields__:
        cp = dataclasses.replace(cp, needs_layout_passes=False)
    return cp


@functools.cache
def _deg_kernel():
    @functools.partial(
        pl.kernel,
        out_type=(jax.ShapeDtypeStruct((_NW, _NPAD), jnp.float32),
                  jax.ShapeDtypeStruct((_NW, _NPAD), jnp.float32)),
        mesh=_mesh(),
        compiler_params=_sc_params(),
        scratch_types=[
            pltpu.VMEM((_NPAD,), jnp.float32),
            pltpu.VMEM((_NPAD,), jnp.float32),
            pltpu.VMEM((_CPW, _CHUNK), jnp.int32),
            pltpu.VMEM((_CPW, _CHUNK), jnp.int32),
        ],
    )
    def deg(src_hbm, dst_hbm, osrc_hbm, odst_hbm, hs, hd, bs, bd):
        wid = lax.axis_index("s") * _NC + lax.axis_index("c")
        zeros = jnp.zeros((_VEC,), jnp.float32)
        ones = jnp.ones((_VEC,), jnp.float32)

        @pl.loop(0, _NPAD, step=_VEC)
        def _(i):
            hs[pl.ds(i, _VEC)] = zeros
            hd[pl.ds(i, _VEC)] = zeros

        row0 = wid * _CPW
        pltpu.sync_copy(src_hbm.at[pl.ds(row0, _CPW)], bs)
        pltpu.sync_copy(dst_hbm.at[pl.ds(row0, _CPW)], bd)

        @pl.loop(0, _CPW)
        def _(j):
            @pl.loop(0, _CHUNK, step=_VEC)
            def _(k):
                plsc.addupdate_scatter(hs, [bs[j, pl.ds(k, _VEC)]], ones)
                plsc.addupdate_scatter(hd, [bd[j, pl.ds(k, _VEC)]], ones)

        pltpu.sync_copy(hs, osrc_hbm.at[wid])
        pltpu.sync_copy(hd, odst_hbm.at[wid])

    return deg


@functools.cache
def _conv_kernel():
    rows_per_sub = _NPAD // _NS

    @functools.partial(
        pl.kernel,
        out_type=(jax.ShapeDtypeStruct((_NPAD, _H), jnp.float32),
                  jax.ShapeDtypeStruct((_NPAD, _H), jnp.float32)),
        mesh=_mesh(),
        compiler_params=_sc_params(),
        scratch_types=[
            pltpu.VMEM_SHARED((_NPAD, _H), jnp.float32),
            pltpu.VMEM((_CPS // 2, _CHUNK), jnp.int32),
            pltpu.VMEM((_CPS // 2, _CHUNK), jnp.int32),
            pltpu.VMEM((_CHUNK, _H), jnp.float32),
            pltpu.VMEM((_CHUNK, _H), jnp.float32),
            pltpu.SemaphoreType.DMA,
            pltpu.SemaphoreType.DMA,
        ],
    )
    def conv(tlo_hbm, thi_hbm, src_hbm, dst_hbm, olo_hbm, ohi_hbm,
             acc, sidx, didx, rows_a, rows_b, sem_a, sem_b):
        c = lax.axis_index("c")
        s = lax.axis_index("s")
        zeros = jnp.zeros((_VEC,), jnp.float32)

        @pl.loop(0, _CHUNK)
        def _(i):
            @pl.loop(0, _H, step=_VEC)
            def _(k):
                rows_a[i, pl.ds(k, _VEC)] = zeros

        @pl.loop(0, rows_per_sub, step=_CHUNK)
        def _(r):
            pltpu.sync_copy(rows_a, acc.at[pl.ds(s * rows_per_sub + r, _CHUNK)])

        plsc.subcore_barrier()

        half = _CPS // 2

        def run(tbl):
            def run_pass(p):
                row0 = s * _CPS + p * half
                pltpu.sync_copy(src_hbm.at[pl.ds(row0, half)], sidx)
                pltpu.sync_copy(dst_hbm.at[pl.ds(row0, half)], didx)

                def wait_a():
                    pltpu.make_async_copy(tbl.at[pl.ds(0, _CHUNK)], rows_a,
                                          sem_a).wait()

                pltpu.async_copy(tbl.at[sidx.at[0]], rows_a, sem_a)

                @pl.loop(0, half, step=2)
                def _(j):
                    cp_b = pltpu.async_copy(tbl.at[sidx.at[j + 1]], rows_b,
                                            sem_b)
                    wait_a()
                    pltpu.sync_copy(rows_a, acc.at[didx.at[j]], add=True)

                    @pl.when(j + 2 < half)
                    def _():
                        pltpu.async_copy(tbl.at[sidx.at[j + 2]], rows_a, sem_a)

                    cp_b.wait()
                    pltpu.sync_copy(rows_b, acc.at[didx.at[j + 1]], add=True)

            run_pass(0)
            run_pass(1)

        @pl.when(c == 0)
        def _():
            run(tlo_hbm)

        @pl.when(c == 1)
        def _():
            run(thi_hbm)

        plsc.subcore_barrier()

        out_slice = pl.ds(s * rows_per_sub, rows_per_sub)

        @pl.when(c == 0)
        def _():
            pltpu.sync_copy(acc.at[out_slice], olo_hbm.at[out_slice])

        @pl.when(c == 1)
        def _():
            pltpu.sync_copy(acc.at[out_slice], ohi_hbm.at[out_slice])

    return conv


@functools.cache
def _edge_kernel():
    @functools.partial(
        pl.kernel,
        out_type=jax.ShapeDtypeStruct((_EROWS, _CHUNK), jnp.float32),
        mesh=_mesh(),
        compiler_params=_sc_params(),
        scratch_types=[
            pltpu.VMEM((_NPAD,), jnp.float32),
            pltpu.VMEM((_NPAD,), jnp.float32),
            pltpu.VMEM((_CPW, _CHUNK), jnp.int32),
            pltpu.VMEM((_CPW, _CHUNK), jnp.int32),
            pltpu.VMEM((_CPW, _CHUNK), jnp.float32),
        ],
    )
    def edge(p_hbm, q_hbm, src_hbm, dst_hbm, o_hbm, pv, qv, bs, bd, ov):
        wid = lax.axis_index("s") * _NC + lax.axis_index("c")
        pltpu.sync_copy(p_hbm, pv)
        pltpu.sync_copy(q_hbm, qv)
        row0 = wid * _CPW
        pltpu.sync_copy(src_hbm.at[pl.ds(row0, _CPW)], bs)
        pltpu.sync_copy(dst_hbm.at[pl.ds(row0, _CPW)], bd)

        @pl.loop(0, _CPW)
        def _(j):
            @pl.loop(0, _CHUNK, step=_VEC)
            def _(k):
                z = (plsc.load_gather(pv, [bs[j, pl.ds(k, _VEC)]])
                     + plsc.load_gather(qv, [bd[j, pl.ds(k, _VEC)]]))
                ov[j, pl.ds(k, _VEC)] = 20.0 / (1.0 + jnp.exp(-z)) - 1.0

        pltpu.sync_copy(ov, o_hbm.at[pl.ds(row0, _CPW)])

    return edge


def _stage_norms(fpad, hsrc, hdst):
    def body(f_ref, hs_ref, hd_ref, tlo_ref, thi_ref, ns_ref, nd_ref):
        od = jnp.sum(hs_ref[...], axis=0)
        idg = jnp.sum(hd_ref[...], axis=0)
        ns = jnp.where(od > 0, od, 1.0) ** -0.5
        nd = jnp.where(idg > 0, idg, 1.0) ** -0.5
        t = f_ref[...] * ns[:, None]
        tlo_ref[...] = t[:, :_H]
        thi_ref[...] = t[:, _H:]
        ns_ref[...] = ns[:, None]
        nd_ref[...] = nd[:, None]

    grid = _NPAD // _BLK
    return pl.pallas_call(
        body,
        grid=(grid,),
        in_specs=[
            pl.BlockSpec((_BLK, _D), lambda i: (i, 0)),
            pl.BlockSpec((_NW, _BLK), lambda i: (0, i)),
            pl.BlockSpec((_NW, _BLK), lambda i: (0, i)),
        ],
        out_specs=[
            pl.BlockSpec((_BLK, _H), lambda i: (i, 0)),
            pl.BlockSpec((_BLK, _H), lambda i: (i, 0)),
            pl.BlockSpec((_BLK, 1), lambda i: (i, 0)),
            pl.BlockSpec((_BLK, 1), lambda i: (i, 0)),
        ],
        out_shape=[
            jax.ShapeDtypeStruct((_NPAD, _H), jnp.float32),
            jax.ShapeDtypeStruct((_NPAD, _H), jnp.float32),
            jax.ShapeDtypeStruct((_NPAD, 1), jnp.float32),
            jax.ShapeDtypeStruct((_NPAD, 1), jnp.float32),
        ],
    )(fpad, hsrc, hdst)


def _stage_mid(alo, ahi, ns, nd, W, b8):
    def body(alo_ref, ahi_ref, ns_ref, nd_ref, w_ref, b_ref, olo_ref, ohi_ref):
        ndv = nd_ref[...]
        xlo = alo_ref[...] * ndv
        xhi = ahi_ref[...] * ndv
        y = (jnp.dot(xlo, w_ref[:_H, :], preferred_element_type=jnp.float32,
                     precision=lax.Precision.HIGHEST)
             + jnp.dot(xhi, w_ref[_H:, :], preferred_element_type=jnp.float32,
                       precision=lax.Precision.HIGHEST))
        h = jnp.maximum(y + b_ref[0:1, :], 0.0)
        t = h * ns_ref[...]
        olo_ref[...] = t[:, :_H]
        ohi_ref[...] = t[:, _H:]

    grid = _NPAD // _BLK
    return pl.pallas_call(
        body,
        grid=(grid,),
        in_specs=[
            pl.BlockSpec((_BLK, _H), lambda i: (i, 0)),
            pl.BlockSpec((_BLK, _H), lambda i: (i, 0)),
            pl.BlockSpec((_BLK, 1), lambda i: (i, 0)),
            pl.BlockSpec((_BLK, 1), lambda i: (i, 0)),
            pl.BlockSpec((_D, _D), lambda i: (0, 0)),
            pl.BlockSpec((8, _D), lambda i: (0, 0)),
        ],
        out_specs=[
            pl.BlockSpec((_BLK, _H), lambda i: (i, 0)),
            pl.BlockSpec((_BLK, _H), lambda i: (i, 0)),
        ],
        out_shape=[
            jax.ShapeDtypeStruct((_NPAD, _H), jnp.float32),
            jax.ShapeDtypeStruct((_NPAD, _H), jnp.float32),
        ],
    )(alo, ahi, ns, nd, W, b8)


def _stage_out(alo, ahi, nd, W, b8, wpq, bout8):
    def body(alo_ref, ahi_ref, nd_ref, w_ref, b_ref, wpq_ref, bo_ref,
             op_ref, oq_ref):
        ndv = nd_ref[...]
        xlo = alo_ref[...] * ndv
        xhi = ahi_ref[...] * ndv
        y = (jnp.dot(xlo, w_ref[:_H, :], preferred_element_type=jnp.float32,
                     precision=lax.Precision.HIGHEST)
             + jnp.dot(xhi, w_ref[_H:, :], preferred_element_type=jnp.float32,
                       precision=lax.Precision.HIGHEST))
        h = jnp.maximum(y + b_ref[0:1, :], 0.0)
        pq = jnp.dot(h, wpq_ref[...], preferred_element_type=jnp.float32,
                     precision=lax.Precision.HIGHEST)
        op_ref[...] = pq[:, 0:1] + bo_ref[0:1, 0:1]
        oq_ref[...] = pq[:, 1:2]

    grid = _NPAD // _BLK
    return pl.pallas_call(
        body,
        grid=(grid,),
        in_specs=[
            pl.BlockSpec((_BLK, _H), lambda i: (i, 0)),
            pl.BlockSpec((_BLK, _H), lambda i: (i, 0)),
            pl.BlockSpec((_BLK, 1), lambda i: (i, 0)),
            pl.BlockSpec((_D, _D), lambda i: (0, 0)),
            pl.BlockSpec((8, _D), lambda i: (0, 0)),
            pl.BlockSpec((_D, 2), lambda i: (0, 0)),
            pl.BlockSpec((8, 128), lambda i: (0, 0)),
        ],
        out_specs=[
            pl.BlockSpec((_BLK, 1), lambda i: (i, 0)),
            pl.BlockSpec((_BLK, 1), lambda i: (i, 0)),
        ],
        out_shape=[
            jax.ShapeDtypeStruct((_NPAD, 1), jnp.float32),
            jax.ShapeDtypeStruct((_NPAD, 1), jnp.float32),
        ],
    )(alo, ahi, nd, W, b8, wpq, bout8)


def kernel(features, edge_index, W1, b1, W2, b2, W_out, b_out):
    src = edge_index[0]
    dst = edge_index[1]
    pad = jnp.full((_EPAD - _E,), _N, dtype=jnp.int32)
    srcp = jnp.concatenate([src, pad]).reshape(_EROWS, _CHUNK)
    dstp = jnp.concatenate([dst, pad]).reshape(_EROWS, _CHUNK)
    fpad = jnp.pad(features, ((0, _NPAD - _N), (0, 0)))

    b1_8 = jnp.broadcast_to(b1[None, :], (8, _D))
    b2_8 = jnp.broadcast_to(b2[None, :], (8, _D))
    wpq = jnp.concatenate([W_out[:_D], W_out[_D:]], axis=1)
    bout8 = jnp.broadcast_to(b_out.reshape(1, 1), (8, 128))

    hsrc, hdst = _deg_kernel()(srcp, dstp)
    t1lo, t1hi, ns, nd = _stage_norms(fpad, hsrc, hdst)
    a1lo, a1hi = _conv_kernel()(t1lo, t1hi, srcp, dstp)
    t2lo, t2hi = _stage_mid(a1lo, a1hi, ns, nd, W1, b1_8)
    a2lo, a2hi = _conv_kernel()(t2lo, t2hi, srcp, dstp)
    p, q = _stage_out(a2lo, a2hi, nd, W2, b2_8, wpq, bout8)

    out = _edge_kernel()(p.reshape(_NPAD), q.reshape(_NPAD), srcp, dstp)
    return out.reshape(_EPAD)[:_E, None]
